# Initial kernel scaffold; baseline (speedup 1.0000x reference)
import jax
import jax.numpy as jnp
from jax.experimental import pallas as pl


def kernel(contributions):
    raise NotImplementedError("write your pallas kernel here")


# speedup vs baseline: 14.6265x; 14.6265x over previous
import functools

import jax
import jax.numpy as jnp
from jax import lax
from jax.experimental import pallas as pl
from jax.experimental.pallas import tpu as pltpu
from jax.experimental.pallas import tpu_sc as plsc

N_TOP = 4
LANES = 16
NUM_CORES = 2
NUM_SUBCORES = 16
NUM_WORKERS = NUM_CORES * NUM_SUBCORES


def _insert(v, m):
    m1, m2, m3, m4 = m
    c1 = jnp.minimum(m1, v)
    m1 = jnp.maximum(m1, v)
    c2 = jnp.minimum(m2, c1)
    m2 = jnp.maximum(m2, c1)
    c3 = jnp.minimum(m3, c2)
    m3 = jnp.maximum(m3, c2)
    m4 = jnp.maximum(m4, c3)
    return (m1, m2, m3, m4)


def kernel(contributions):
    n_samples, n_proto, n_class = contributions.shape
    n_cgroups = n_class // LANES
    n_tasks = n_samples * n_cgroups
    tasks_per_worker = n_tasks // NUM_WORKERS
    cg_shift = n_cgroups.bit_length() - 1

    mesh = plsc.VectorSubcoreMesh(core_axis_name="c", subcore_axis_name="s")

    @functools.partial(
        pl.kernel,
        mesh=mesh,
        out_type=jax.ShapeDtypeStruct(contributions.shape, contributions.dtype),
        compiler_params=pltpu.CompilerParams(use_tc_tiling_on_sc=False),
        scratch_types=[
            pltpu.VMEM((n_proto, LANES), jnp.float32),
            pltpu.SemaphoreType.DMA,
        ],
    )
    def _run(x_hbm, out_hbm, tile, sem):
        wid = lax.axis_index("s") * NUM_CORES + lax.axis_index("c")

        @pl.loop(0, tasks_per_worker)
        def _task(t):
            tg = wid * tasks_per_worker + t
            s_idx = tg >> cg_shift
            c0 = (tg & (n_cgroups - 1)) * LANES

            pltpu.async_copy(
                x_hbm.at[s_idx, :, pl.ds(c0, LANES)], tile, sem
            ).wait()

            neg_inf = jnp.full((LANES,), -jnp.inf, jnp.float32)
            init = (neg_inf,) * (4 * N_TOP)

            def body(i, acc):
                sets = [list(acc[4 * k : 4 * k + 4]) for k in range(4)]
                for k in range(4):
                    v = tile[i * 4 + k]
                    sets[k] = list(_insert(v, tuple(sets[k])))
                return tuple(x for st in sets for x in st)

            acc = lax.fori_loop(0, n_proto // 4, body, init)
            top = tuple(acc[0:4])
            for k in range(1, 4):
                for j in range(4):
                    top = _insert(acc[4 * k + j], top)
            thresh = top[3]

            zeros = jnp.zeros((LANES,), jnp.float32)

            @pl.loop(0, n_proto, step=8)
            def _mask(p):
                for k in range(8):
                    v = tile[p + k]
                    tile[p + k] = jnp.where(v >= thresh, zeros, v)

            pltpu.async_copy(
                tile, out_hbm.at[s_idx, :, pl.ds(c0, LANES)], sem
            ).wait()

    return _run(contributions)


# speedup vs baseline: 17.6237x; 1.2049x over previous
import functools

import jax
import jax.numpy as jnp
from jax import lax
from jax.experimental import pallas as pl
from jax.experimental.pallas import tpu as pltpu
from jax.experimental.pallas import tpu_sc as plsc

N_TOP = 4
LANES = 16
NUM_CORES = 2
NUM_SUBCORES = 16
NUM_WORKERS = NUM_CORES * NUM_SUBCORES


def _insert(v, m):
    m1, m2, m3, m4 = m
    c1 = jnp.minimum(m1, v)
    m1 = jnp.maximum(m1, v)
    c2 = jnp.minimum(m2, c1)
    m2 = jnp.maximum(m2, c1)
    c3 = jnp.minimum(m3, c2)
    m3 = jnp.maximum(m3, c2)
    m4 = jnp.maximum(m4, c3)
    return (m1, m2, m3, m4)


def kernel(contributions):
    n_samples, n_proto, n_class = contributions.shape
    n_cgroups = n_class // LANES
    n_tasks = n_samples * n_cgroups
    tasks_per_worker = n_tasks // NUM_WORKERS
    cg_shift = n_cgroups.bit_length() - 1

    mesh = plsc.VectorSubcoreMesh(core_axis_name="c", subcore_axis_name="s")

    def _compute(tile):
        neg_inf = jnp.full((LANES,), -jnp.inf, jnp.float32)
        init = (neg_inf,) * (4 * N_TOP)

        def body(i, acc):
            sets = [list(acc[4 * k : 4 * k + 4]) for k in range(4)]
            for k in range(4):
                v = tile[i * 4 + k]
                sets[k] = list(_insert(v, tuple(sets[k])))
            return tuple(x for st in sets for x in st)

        acc = lax.fori_loop(0, n_proto // 4, body, init)
        top = tuple(acc[0:4])
        for k in range(1, 4):
            for j in range(4):
                top = _insert(acc[4 * k + j], top)
        thresh = top[3]

        zeros = jnp.zeros((LANES,), jnp.float32)

        @pl.loop(0, n_proto, step=8)
        def _mask(p):
            for k in range(8):
                v = tile[p + k]
                tile[p + k] = jnp.where(v >= thresh, zeros, v)

    @functools.partial(
        pl.kernel,
        mesh=mesh,
        out_type=jax.ShapeDtypeStruct(contributions.shape, contributions.dtype),
        compiler_params=pltpu.CompilerParams(use_tc_tiling_on_sc=False),
        scratch_types=[
            pltpu.VMEM((n_proto, LANES), jnp.float32),
            pltpu.VMEM((n_proto, LANES), jnp.float32),
            pltpu.SemaphoreType.DMA,
            pltpu.SemaphoreType.DMA,
            pltpu.SemaphoreType.DMA,
            pltpu.SemaphoreType.DMA,
        ],
    )
    def _run(x_hbm, out_hbm, buf0, buf1, l0, l1, s0, s1):
        wid = lax.axis_index("s") * NUM_CORES + lax.axis_index("c")
        bufs = (buf0, buf1)
        lsems = (l0, l1)
        ssems = (s0, s1)

        def _slc(hbm, tc):
            tg = wid * tasks_per_worker + tc
            s_idx = tg >> cg_shift
            c0 = (tg & (n_cgroups - 1)) * LANES
            return hbm.at[s_idx, :, pl.ds(c0, LANES)]

        def load(tc, b):
            return pltpu.make_async_copy(_slc(x_hbm, tc), bufs[b], lsems[b])

        def store(tc, b):
            return pltpu.make_async_copy(bufs[b], _slc(out_hbm, tc), ssems[b])

        load(0, 0).start()

        @pl.loop(0, tasks_per_worker, step=2)
        def _pair(t):
            for par in range(2):
                cur, nxt = par, 1 - par
                tc = t + par
                load(tc, cur).wait()

                @pl.when(tc + 1 < tasks_per_worker)
                def _prefetch():
                    @pl.when(tc >= 1)
                    def _drain():
                        store(tc - 1, nxt).wait()

                    load(tc + 1, nxt).start()

                _compute(bufs[cur])
                store(tc, cur).start()

        store(tasks_per_worker - 2, 0).wait()
        store(tasks_per_worker - 1, 1).wait()

    return _run(contributions)


# speedup vs baseline: 17.7035x; 1.0045x over previous
import functools

import jax
import jax.numpy as jnp
from jax import lax
from jax.experimental import pallas as pl
from jax.experimental.pallas import tpu as pltpu
from jax.experimental.pallas import tpu_sc as plsc

N_TOP = 4
LANES = 16
NUM_CORES = 2
NUM_SUBCORES = 16
NUM_WORKERS = NUM_CORES * NUM_SUBCORES
CHUNK_ROWS = 128
N_SETS = 4


def _insert(v, m):
    m1, m2, m3, m4 = m
    c1 = jnp.minimum(m1, v)
    m1 = jnp.maximum(m1, v)
    c2 = jnp.minimum(m2, c1)
    m2 = jnp.maximum(m2, c1)
    c3 = jnp.minimum(m3, c2)
    m3 = jnp.maximum(m3, c2)
    m4 = jnp.maximum(m4, c3)
    return (m1, m2, m3, m4)


def kernel(contributions):
    n_samples, n_proto, n_class = contributions.shape
    n_groups = n_class // LANES
    n_chunks = n_proto // CHUNK_ROWS
    samples_per_worker = n_samples // NUM_WORKERS
    acc_rows = n_groups * N_SETS * N_TOP

    mesh = plsc.VectorSubcoreMesh(core_axis_name="c", subcore_axis_name="s")

    @functools.partial(
        pl.kernel,
        mesh=mesh,
        out_type=jax.ShapeDtypeStruct(contributions.shape, contributions.dtype),
        compiler_params=pltpu.CompilerParams(use_tc_tiling_on_sc=False),
        scratch_types=[
            pltpu.VMEM((CHUNK_ROWS, n_class), jnp.float32),
            pltpu.VMEM((CHUNK_ROWS, n_class), jnp.float32),
            pltpu.VMEM((acc_rows, LANES), jnp.float32),
            pltpu.VMEM((n_groups, LANES), jnp.float32),
            pltpu.SemaphoreType.DMA,
            pltpu.SemaphoreType.DMA,
            pltpu.SemaphoreType.DMA,
            pltpu.SemaphoreType.DMA,
        ],
    )
    def _run(x_hbm, out_hbm, buf0, buf1, acc, thresh, l0, l1, s0, s1):
        wid = lax.axis_index("s") * NUM_CORES + lax.axis_index("c")
        bufs = (buf0, buf1)
        lsems = (l0, l1)
        ssems = (s0, s1)

        def load(s_idx, ck, b):
            return pltpu.make_async_copy(
                x_hbm.at[s_idx, pl.ds(ck * CHUNK_ROWS, CHUNK_ROWS), :],
                bufs[b],
                lsems[b],
            )

        def store(s_idx, ck, b):
            return pltpu.make_async_copy(
                bufs[b],
                out_hbm.at[s_idx, pl.ds(ck * CHUNK_ROWS, CHUNK_ROWS), :],
                ssems[b],
            )

        def accumulate(tile):
            @pl.loop(0, n_groups)
            def _grp(j):
                a0 = j * (N_SETS * N_TOP)
                sets = [
                    [acc[a0 + 4 * k + i] for i in range(N_TOP)]
                    for k in range(N_SETS)
                ]

                def body(i, flat):
                    st = [list(flat[4 * k : 4 * k + 4]) for k in range(N_SETS)]
                    for k in range(N_SETS):
                        v = tile[i * N_SETS + k, pl.ds(j * LANES, LANES)]
                        st[k] = list(_insert(v, tuple(st[k])))
                    return tuple(x for s_ in st for x in s_)

                flat = lax.fori_loop(
                    0,
                    CHUNK_ROWS // N_SETS,
                    body,
                    tuple(x for s_ in sets for x in s_),
                )
                for i in range(N_SETS * N_TOP):
                    acc[a0 + i] = flat[i]

        def finalize():
            @pl.loop(0, n_groups)
            def _grp(j):
                a0 = j * (N_SETS * N_TOP)
                top = tuple(acc[a0 + i] for i in range(N_TOP))
                for k in range(1, N_SETS):
                    for i in range(N_TOP):
                        top = _insert(acc[a0 + 4 * k + i], top)
                thresh[j] = top[3]

        def mask(tile):
            zeros = jnp.zeros((LANES,), jnp.float32)

            @pl.loop(0, n_groups)
            def _grp(j):
                thr = thresh[j]

                @pl.loop(0, CHUNK_ROWS, step=8)
                def _rows(r):
                    for k in range(8):
                        v = tile[r + k, pl.ds(j * LANES, LANES)]
                        tile[r + k, pl.ds(j * LANES, LANES)] = jnp.where(
                            v >= thr, zeros, v
                        )

        neg_inf = jnp.full((LANES,), -jnp.inf, jnp.float32)

        for si in range(samples_per_worker):
            s_idx = wid * samples_per_worker + si

            @pl.loop(0, acc_rows)
            def _init(g):
                acc[g] = neg_inf

            load(s_idx, 0, 0).start()

            @pl.loop(0, n_chunks, step=2)
            def _pa(t):
                for par in range(2):
                    cur, nxt = par, 1 - par
                    ck = t + par
                    load(s_idx, ck, cur).wait()

                    @pl.when(ck + 1 < n_chunks)
                    def _pf():
                        load(s_idx, ck + 1, nxt).start()

                    accumulate(bufs[cur])

            load(s_idx, 0, 0).start()
            finalize()

            @pl.loop(0, n_chunks, step=2)
            def _pb(t):
                for par in range(2):
                    cur, nxt = par, 1 - par
                    ck = t + par
                    load(s_idx, ck, cur).wait()

                    @pl.when(ck + 1 < n_chunks)
                    def _pf():
                        @pl.when(ck >= 1)
                        def _drain():
                            store(s_idx, ck - 1, nxt).wait()

                        load(s_idx, ck + 1, nxt).start()

                    mask(bufs[cur])
                    store(s_idx, ck, cur).start()

            store(s_idx, n_chunks - 2, 0).wait()
            store(s_idx, n_chunks - 1, 1).wait()

    return _run(contributions)


# speedup vs baseline: 36.1261x; 2.0406x over previous
import functools

import jax
import jax.numpy as jnp
from jax import lax
from jax.experimental import pallas as pl
from jax.experimental.pallas import tpu as pltpu
from jax.experimental.pallas import tpu_sc as plsc

N_TOP = 4
LANES = 16
NUM_CORES = 2
NUM_SUBCORES = 16
NUM_WORKERS = NUM_CORES * NUM_SUBCORES
CHUNK_ROWS = 128
N_SETS = 4


def _insert(v, m):
    m1, m2, m3, m4 = m
    c1 = jnp.minimum(m1, v)
    m1 = jnp.maximum(m1, v)
    c2 = jnp.minimum(m2, c1)
    m2 = jnp.maximum(m2, c1)
    c3 = jnp.minimum(m3, c2)
    m3 = jnp.maximum(m3, c2)
    m4 = jnp.maximum(m4, c3)
    return (m1, m2, m3, m4)


def kernel(contributions):
    n_samples, n_proto, n_class = contributions.shape
    n_groups = n_class // LANES
    n_chunks = n_proto // CHUNK_ROWS
    samples_per_worker = n_samples // NUM_WORKERS
    acc_rows = n_groups * N_SETS * N_TOP

    mesh = plsc.VectorSubcoreMesh(core_axis_name="c", subcore_axis_name="s")

    @functools.partial(
        pl.kernel,
        mesh=mesh,
        out_type=jax.ShapeDtypeStruct(contributions.shape, contributions.dtype),
        compiler_params=pltpu.CompilerParams(use_tc_tiling_on_sc=True),
        scratch_types=[
            pltpu.VMEM((CHUNK_ROWS, n_class), jnp.float32),
            pltpu.VMEM((CHUNK_ROWS, n_class), jnp.float32),
            pltpu.VMEM((acc_rows, LANES), jnp.float32),
            pltpu.VMEM((n_groups, LANES), jnp.float32),
            pltpu.SemaphoreType.DMA,
            pltpu.SemaphoreType.DMA,
            pltpu.SemaphoreType.DMA,
            pltpu.SemaphoreType.DMA,
        ],
    )
    def _run(x_hbm, out_hbm, buf0, buf1, acc, thresh, l0, l1, s0, s1):
        wid = lax.axis_index("s") * NUM_CORES + lax.axis_index("c")
        bufs = (buf0, buf1)
        lsems = (l0, l1)
        ssems = (s0, s1)

        def load(s_idx, ck, b):
            return pltpu.make_async_copy(
                x_hbm.at[s_idx, pl.ds(ck * CHUNK_ROWS, CHUNK_ROWS), :],
                bufs[b],
                lsems[b],
            )

        def store(s_idx, ck, b):
            return pltpu.make_async_copy(
                bufs[b],
                out_hbm.at[s_idx, pl.ds(ck * CHUNK_ROWS, CHUNK_ROWS), :],
                ssems[b],
            )

        def accumulate(tile):
            @pl.loop(0, n_groups)
            def _grp(j):
                a0 = j * (N_SETS * N_TOP)
                sets = [
                    [acc[a0 + 4 * k + i] for i in range(N_TOP)]
                    for k in range(N_SETS)
                ]

                def body(i, flat):
                    st = [list(flat[4 * k : 4 * k + 4]) for k in range(N_SETS)]
                    for k in range(N_SETS):
                        v = tile[i * N_SETS + k, pl.ds(j * LANES, LANES)]
                        st[k] = list(_insert(v, tuple(st[k])))
                    return tuple(x for s_ in st for x in s_)

                flat = lax.fori_loop(
                    0,
                    CHUNK_ROWS // N_SETS,
                    body,
                    tuple(x for s_ in sets for x in s_),
                )
                for i in range(N_SETS * N_TOP):
                    acc[a0 + i] = flat[i]

        def finalize():
            @pl.loop(0, n_groups)
            def _grp(j):
                a0 = j * (N_SETS * N_TOP)
                top = tuple(acc[a0 + i] for i in range(N_TOP))
                for k in range(1, N_SETS):
                    for i in range(N_TOP):
                        top = _insert(acc[a0 + 4 * k + i], top)
                thresh[j] = top[3]

        def mask(tile):
            zeros = jnp.zeros((LANES,), jnp.float32)

            @pl.loop(0, n_groups)
            def _grp(j):
                thr = thresh[j]

                @pl.loop(0, CHUNK_ROWS, step=8)
                def _rows(r):
                    for k in range(8):
                        v = tile[r + k, pl.ds(j * LANES, LANES)]
                        tile[r + k, pl.ds(j * LANES, LANES)] = jnp.where(
                            v >= thr, zeros, v
                        )

        neg_inf = jnp.full((LANES,), -jnp.inf, jnp.float32)

        for si in range(samples_per_worker):
            s_idx = wid * samples_per_worker + si

            @pl.loop(0, acc_rows)
            def _init(g):
                acc[g] = neg_inf

            load(s_idx, 0, 0).start()

            @pl.loop(0, n_chunks, step=2)
            def _pa(t):
                for par in range(2):
                    cur, nxt = par, 1 - par
                    ck = t + par
                    load(s_idx, ck, cur).wait()

                    @pl.when(ck + 1 < n_chunks)
                    def _pf():
                        load(s_idx, ck + 1, nxt).start()

                    accumulate(bufs[cur])

            load(s_idx, 0, 0).start()
            finalize()

            @pl.loop(0, n_chunks, step=2)
            def _pb(t):
                for par in range(2):
                    cur, nxt = par, 1 - par
                    ck = t + par
                    load(s_idx, ck, cur).wait()

                    @pl.when(ck + 1 < n_chunks)
                    def _pf():
                        @pl.when(ck >= 1)
                        def _drain():
                            store(s_idx, ck - 1, nxt).wait()

                        load(s_idx, ck + 1, nxt).start()

                    mask(bufs[cur])
                    store(s_idx, ck, cur).start()

            store(s_idx, n_chunks - 2, 0).wait()
            store(s_idx, n_chunks - 1, 1).wait()

    return _run(contributions)


# speedup vs baseline: 40.1967x; 1.1127x over previous
import functools

import jax
import jax.numpy as jnp
from jax import lax
from jax.experimental import pallas as pl
from jax.experimental.pallas import tpu as pltpu
from jax.experimental.pallas import tpu_sc as plsc

N_TOP = 4
LANES = 16
NUM_CORES = 2
NUM_SUBCORES = 16
NUM_WORKERS = NUM_CORES * NUM_SUBCORES
CHUNK_ROWS = 128
N_SETS = 4


def _sort4(v0, v1, v2, v3):
    a0, a1 = jnp.maximum(v0, v1), jnp.minimum(v0, v1)
    a2, a3 = jnp.maximum(v2, v3), jnp.minimum(v2, v3)
    b0, b2 = jnp.maximum(a0, a2), jnp.minimum(a0, a2)
    b1, b3 = jnp.maximum(a1, a3), jnp.minimum(a1, a3)
    c1, c2 = jnp.maximum(b1, b2), jnp.minimum(b1, b2)
    return b0, c1, c2, b3


def _merge4(a, b):
    a1, a2, a3, a4 = a
    b1, b2, b3, b4 = b
    l1 = jnp.maximum(a1, b4)
    l2 = jnp.maximum(a2, b3)
    l3 = jnp.maximum(a3, b2)
    l4 = jnp.maximum(a4, b1)
    m1, m3 = jnp.maximum(l1, l3), jnp.minimum(l1, l3)
    m2, m4 = jnp.maximum(l2, l4), jnp.minimum(l2, l4)
    r1, r2 = jnp.maximum(m1, m2), jnp.minimum(m1, m2)
    r3, r4 = jnp.maximum(m3, m4), jnp.minimum(m3, m4)
    return r1, r2, r3, r4


def kernel(contributions):
    n_samples, n_proto, n_class = contributions.shape
    n_groups = n_class // LANES
    n_chunks = n_proto // CHUNK_ROWS
    samples_per_worker = n_samples // NUM_WORKERS
    acc_rows = n_groups * N_SETS * N_TOP

    mesh = plsc.VectorSubcoreMesh(core_axis_name="c", subcore_axis_name="s")

    @functools.partial(
        pl.kernel,
        mesh=mesh,
        out_type=jax.ShapeDtypeStruct(contributions.shape, contributions.dtype),
        compiler_params=pltpu.CompilerParams(use_tc_tiling_on_sc=True),
        scratch_types=[
            pltpu.VMEM((CHUNK_ROWS, n_class), jnp.float32),
            pltpu.VMEM((CHUNK_ROWS, n_class), jnp.float32),
            pltpu.VMEM((acc_rows, LANES), jnp.float32),
            pltpu.VMEM((n_groups, LANES), jnp.float32),
            pltpu.SemaphoreType.DMA,
            pltpu.SemaphoreType.DMA,
            pltpu.SemaphoreType.DMA,
            pltpu.SemaphoreType.DMA,
        ],
    )
    def _run(x_hbm, out_hbm, buf0, buf1, acc, thresh, l0, l1, s0, s1):
        wid = lax.axis_index("s") * NUM_CORES + lax.axis_index("c")
        bufs = (buf0, buf1)
        lsems = (l0, l1)
        ssems = (s0, s1)

        def load(s_idx, ck, b):
            return pltpu.make_async_copy(
                x_hbm.at[s_idx, pl.ds(ck * CHUNK_ROWS, CHUNK_ROWS), :],
                bufs[b],
                lsems[b],
            )

        def store(s_idx, ck, b):
            return pltpu.make_async_copy(
                bufs[b],
                out_hbm.at[s_idx, pl.ds(ck * CHUNK_ROWS, CHUNK_ROWS), :],
                ssems[b],
            )

        def accumulate(tile):
            @pl.loop(0, n_groups)
            def _grp(j):
                a0 = j * (N_SETS * N_TOP)
                sets = [
                    [acc[a0 + 4 * k + i] for i in range(N_TOP)]
                    for k in range(N_SETS)
                ]

                def body(i, flat):
                    st = [list(flat[4 * k : 4 * k + 4]) for k in range(N_SETS)]
                    for k in range(N_SETS):
                        r0 = i * (4 * N_SETS) + 4 * k
                        rows = _sort4(
                            *(
                                tile[r0 + d, pl.ds(j * LANES, LANES)]
                                for d in range(4)
                            )
                        )
                        st[k] = list(_merge4(tuple(st[k]), rows))
                    return tuple(x for s_ in st for x in s_)

                flat = lax.fori_loop(
                    0,
                    CHUNK_ROWS // (4 * N_SETS),
                    body,
                    tuple(x for s_ in sets for x in s_),
                )
                for i in range(N_SETS * N_TOP):
                    acc[a0 + i] = flat[i]

        def finalize():
            @pl.loop(0, n_groups)
            def _grp(j):
                a0 = j * (N_SETS * N_TOP)
                sets = [
                    tuple(acc[a0 + 4 * k + i] for i in range(N_TOP))
                    for k in range(N_SETS)
                ]
                top = _merge4(
                    _merge4(sets[0], sets[1]), _merge4(sets[2], sets[3])
                )
                thresh[j] = top[3]

        def mask(tile):
            zeros = jnp.zeros((LANES,), jnp.float32)

            @pl.loop(0, n_groups)
            def _grp(j):
                thr = thresh[j]

                @pl.loop(0, CHUNK_ROWS, step=8)
                def _rows(r):
                    for k in range(8):
                        v = tile[r + k, pl.ds(j * LANES, LANES)]
                        tile[r + k, pl.ds(j * LANES, LANES)] = jnp.where(
                            v >= thr, zeros, v
                        )

        neg_inf = jnp.full((LANES,), -jnp.inf, jnp.float32)

        for si in range(samples_per_worker):
            s_idx = wid * samples_per_worker + si

            @pl.loop(0, acc_rows)
            def _init(g):
                acc[g] = neg_inf

            load(s_idx, 0, 0).start()

            @pl.loop(0, n_chunks, step=2)
            def _pa(t):
                for par in range(2):
                    cur, nxt = par, 1 - par
                    ck = t + par
                    load(s_idx, ck, cur).wait()

                    @pl.when(ck + 1 < n_chunks)
                    def _pf():
                        load(s_idx, ck + 1, nxt).start()

                    accumulate(bufs[cur])

            load(s_idx, 0, 0).start()
            finalize()

            @pl.loop(0, n_chunks, step=2)
            def _pb(t):
                for par in range(2):
                    cur, nxt = par, 1 - par
                    ck = t + par
                    load(s_idx, ck, cur).wait()

                    @pl.when(ck + 1 < n_chunks)
                    def _pf():
                        @pl.when(ck >= 1)
                        def _drain():
                            store(s_idx, ck - 1, nxt).wait()

                        load(s_idx, ck + 1, nxt).start()

                    mask(bufs[cur])
                    store(s_idx, ck, cur).start()

            store(s_idx, n_chunks - 2, 0).wait()
            store(s_idx, n_chunks - 1, 1).wait()

    return _run(contributions)


# speedup vs baseline: 41.8608x; 1.0414x over previous
import functools

import jax
import jax.numpy as jnp
from jax import lax
from jax.experimental import pallas as pl
from jax.experimental.pallas import tpu as pltpu
from jax.experimental.pallas import tpu_sc as plsc

N_TOP = 4
LANES = 16
NUM_CORES = 2
NUM_SUBCORES = 16
NUM_WORKERS = NUM_CORES * NUM_SUBCORES
CHUNK_ROWS = 64
N_SETS = 4


def _sort4(v0, v1, v2, v3):
    a0, a1 = jnp.maximum(v0, v1), jnp.minimum(v0, v1)
    a2, a3 = jnp.maximum(v2, v3), jnp.minimum(v2, v3)
    b0, b2 = jnp.maximum(a0, a2), jnp.minimum(a0, a2)
    b1, b3 = jnp.maximum(a1, a3), jnp.minimum(a1, a3)
    c1, c2 = jnp.maximum(b1, b2), jnp.minimum(b1, b2)
    return b0, c1, c2, b3


def _merge4(a, b):
    a1, a2, a3, a4 = a
    b1, b2, b3, b4 = b
    l1 = jnp.maximum(a1, b4)
    l2 = jnp.maximum(a2, b3)
    l3 = jnp.maximum(a3, b2)
    l4 = jnp.maximum(a4, b1)
    m1, m3 = jnp.maximum(l1, l3), jnp.minimum(l1, l3)
    m2, m4 = jnp.maximum(l2, l4), jnp.minimum(l2, l4)
    r1, r2 = jnp.maximum(m1, m2), jnp.minimum(m1, m2)
    r3, r4 = jnp.maximum(m3, m4), jnp.minimum(m3, m4)
    return r1, r2, r3, r4


def kernel(contributions):
    n_samples, n_proto, n_class = contributions.shape
    n_groups = n_class // LANES
    n_chunks = n_proto // CHUNK_ROWS
    samples_per_worker = n_samples // NUM_WORKERS
    acc_rows = n_groups * N_SETS * N_TOP

    mesh = plsc.VectorSubcoreMesh(core_axis_name="c", subcore_axis_name="s")

    @functools.partial(
        pl.kernel,
        mesh=mesh,
        out_type=jax.ShapeDtypeStruct(contributions.shape, contributions.dtype),
        compiler_params=pltpu.CompilerParams(use_tc_tiling_on_sc=True),
        scratch_types=[
            pltpu.VMEM((CHUNK_ROWS, n_class), jnp.float32),
            pltpu.VMEM((CHUNK_ROWS, n_class), jnp.float32),
            pltpu.VMEM((CHUNK_ROWS, n_class), jnp.float32),
            pltpu.VMEM((CHUNK_ROWS, n_class), jnp.float32),
            pltpu.VMEM((acc_rows, LANES), jnp.float32),
            pltpu.VMEM((n_groups, LANES), jnp.float32),
            pltpu.SemaphoreType.DMA,
            pltpu.SemaphoreType.DMA,
            pltpu.SemaphoreType.DMA,
            pltpu.SemaphoreType.DMA,
            pltpu.SemaphoreType.DMA,
            pltpu.SemaphoreType.DMA,
        ],
    )
    def _run(
        x_hbm, out_hbm, a0, a1, b0, b1, acc, thresh, la0, la1, lb0, lb1, sb0, sb1
    ):
        wid = lax.axis_index("s") * NUM_CORES + lax.axis_index("c")
        bufs_a = (a0, a1)
        bufs_b = (b0, b1)
        lsems_a = (la0, la1)
        lsems_b = (lb0, lb1)
        ssems = (sb0, sb1)

        def load_a(s_idx, ck, b):
            return pltpu.make_async_copy(
                x_hbm.at[s_idx, pl.ds(ck * CHUNK_ROWS, CHUNK_ROWS), :],
                bufs_a[b],
                lsems_a[b],
            )

        def load_b(s_idx, ck, b):
            return pltpu.make_async_copy(
                x_hbm.at[s_idx, pl.ds(ck * CHUNK_ROWS, CHUNK_ROWS), :],
                bufs_b[b],
                lsems_b[b],
            )

        def store(s_idx, ck, b):
            return pltpu.make_async_copy(
                bufs_b[b],
                out_hbm.at[s_idx, pl.ds(ck * CHUNK_ROWS, CHUNK_ROWS), :],
                ssems[b],
            )

        def accumulate(tile):
            @pl.loop(0, n_groups)
            def _grp(j):
                a0 = j * (N_SETS * N_TOP)
                sets = [
                    [acc[a0 + 4 * k + i] for i in range(N_TOP)]
                    for k in range(N_SETS)
                ]

                def body(i, flat):
                    st = [list(flat[4 * k : 4 * k + 4]) for k in range(N_SETS)]
                    for k in range(N_SETS):
                        r0 = i * (4 * N_SETS) + 4 * k
                        rows = _sort4(
                            *(
                                tile[r0 + d, pl.ds(j * LANES, LANES)]
                                for d in range(4)
                            )
                        )
                        st[k] = list(_merge4(tuple(st[k]), rows))
                    return tuple(x for s_ in st for x in s_)

                flat = lax.fori_loop(
                    0,
                    CHUNK_ROWS // (4 * N_SETS),
                    body,
                    tuple(x for s_ in sets for x in s_),
                )
                for i in range(N_SETS * N_TOP):
                    acc[a0 + i] = flat[i]

        def finalize():
            @pl.loop(0, n_groups)
            def _grp(j):
                a0 = j * (N_SETS * N_TOP)
                sets = [
                    tuple(acc[a0 + 4 * k + i] for i in range(N_TOP))
                    for k in range(N_SETS)
                ]
                top = _merge4(
                    _merge4(sets[0], sets[1]), _merge4(sets[2], sets[3])
                )
                thresh[j] = top[3]

        def mask(tile):
            zeros = jnp.zeros((LANES,), jnp.float32)

            @pl.loop(0, n_groups)
            def _grp(j):
                thr = thresh[j]

                @pl.loop(0, CHUNK_ROWS, step=8)
                def _rows(r):
                    for k in range(8):
                        v = tile[r + k, pl.ds(j * LANES, LANES)]
                        tile[r + k, pl.ds(j * LANES, LANES)] = jnp.where(
                            v >= thr, zeros, v
                        )

        neg_inf = jnp.full((LANES,), -jnp.inf, jnp.float32)

        def init_acc():
            @pl.loop(0, acc_rows)
            def _init(g):
                acc[g] = neg_inf

        s0_idx = wid * samples_per_worker
        s1_idx = s0_idx + 1

        init_acc()
        load_a(s0_idx, 0, 0).start()

        @pl.loop(0, n_chunks, step=2)
        def _s1loop(t):
            for par in range(2):
                cur, nxt = par, 1 - par
                ck = t + par
                load_a(s0_idx, ck, cur).wait()

                @pl.when(ck + 1 < n_chunks)
                def _pf():
                    load_a(s0_idx, ck + 1, nxt).start()

                accumulate(bufs_a[cur])

        load_b(s0_idx, 0, 0).start()
        load_a(s1_idx, 0, 0).start()
        finalize()
        init_acc()

        @pl.loop(0, n_chunks, step=2)
        def _s2loop(t):
            for par in range(2):
                cur, nxt = par, 1 - par
                ck = t + par
                load_b(s0_idx, ck, cur).wait()

                @pl.when(ck + 1 < n_chunks)
                def _pfb():
                    @pl.when(ck >= 1)
                    def _drain():
                        store(s0_idx, ck - 1, nxt).wait()

                    load_b(s0_idx, ck + 1, nxt).start()

                load_a(s1_idx, ck, cur).wait()

                @pl.when(ck + 1 < n_chunks)
                def _pfa():
                    load_a(s1_idx, ck + 1, nxt).start()

                mask(bufs_b[cur])
                store(s0_idx, ck, cur).start()
                accumulate(bufs_a[cur])

        store(s0_idx, n_chunks - 2, 0).wait()
        store(s0_idx, n_chunks - 1, 1).wait()
        load_b(s1_idx, 0, 0).start()
        finalize()

        @pl.loop(0, n_chunks, step=2)
        def _s3loop(t):
            for par in range(2):
                cur, nxt = par, 1 - par
                ck = t + par
                load_b(s1_idx, ck, cur).wait()

                @pl.when(ck + 1 < n_chunks)
                def _pf():
                    @pl.when(ck >= 1)
                    def _drain():
                        store(s1_idx, ck - 1, nxt).wait()

                    load_b(s1_idx, ck + 1, nxt).start()

                mask(bufs_b[cur])
                store(s1_idx, ck, cur).start()

        store(s1_idx, n_chunks - 2, 0).wait()
        store(s1_idx, n_chunks - 1, 1).wait()

    return _run(contributions)
